# Initial kernel scaffold; baseline (speedup 1.0000x reference)
import functools

import jax
import jax.numpy as jnp
from jax import lax
from jax.experimental import pallas as pl
from jax.experimental.pallas import tpu as pltpu
from jax.experimental.pallas import tpu_sc as plsc

N_NODES = 100000
N_EDGES = 3200000
NC = 2
NS = 16
NW = NC * NS
CHUNK = 512
SUB = 128
PER_W = 100352
EP = NW * PER_W
NCHUNK = PER_W // CHUNK
NP = 100016
ROWS_PER_TILE = NP // NS
DTAB = 48

F32 = jnp.float32


def _mesh():
    return plsc.VectorSubcoreMesh(
        core_axis_name="c", subcore_axis_name="s", num_cores=NC, num_subcores=NS
    )


def _gather_body(t_hbm, u_hbm, dst2d, src2d, g1_out, g2_out,
                 idxd, idxs, r1, r2, sem):
    wid = lax.axis_index("c") * NS + lax.axis_index("s")
    row_base = wid * (PER_W // SUB)
    e_base = wid * PER_W

    def chunk(i, carry):
        row0 = row_base + i * (CHUNK // SUB)
        pltpu.sync_copy(dst2d.at[pl.ds(row0, CHUNK // SUB)], idxd)
        pltpu.sync_copy(src2d.at[pl.ds(row0, CHUNK // SUB)], idxs)
        copies = []
        for j in range(CHUNK // SUB):
            copies.append(pltpu.async_copy(
                t_hbm.at[idxd.at[j]], r1.at[pl.ds(j * SUB, SUB)], sem))
            copies.append(pltpu.async_copy(
                u_hbm.at[idxs.at[j]], r2.at[pl.ds(j * SUB, SUB)], sem))
        for cp in copies:
            cp.wait()
        e0 = e_base + i * CHUNK
        pltpu.sync_copy(r1, g1_out.at[pl.ds(e0, CHUNK)])
        pltpu.sync_copy(r2, g2_out.at[pl.ds(e0, CHUNK)])
        return carry

    lax.fori_loop(0, NCHUNK, chunk, 0)


def _make_gather():
    return pl.kernel(
        _gather_body,
        out_type=[
            jax.ShapeDtypeStruct((EP, DTAB), F32),
            jax.ShapeDtypeStruct((EP, DTAB), F32),
        ],
        mesh=_mesh(),
        scratch_types=[
            pltpu.VMEM((CHUNK // SUB, SUB), jnp.int32),
            pltpu.VMEM((CHUNK // SUB, SUB), jnp.int32),
            pltpu.VMEM((CHUNK, DTAB), F32),
            pltpu.VMEM((CHUNK, DTAB), F32),
            pltpu.SemaphoreType.DMA,
        ],
    )


def _scatter_body(g_hbm, dst2d, out, acc, zbuf, rows, idxd):
    c = lax.axis_index("c")
    s = lax.axis_index("s")
    wid = c * NS + s

    def zrow(i, carry):
        zbuf[i, :] = jnp.zeros((16,), F32)
        return carry
    lax.fori_loop(0, SUB, zrow, 0)
    zbase = s * ROWS_PER_TILE
    nfull = ROWS_PER_TILE // SUB
    rem = ROWS_PER_TILE - nfull * SUB

    def zcopy(k, carry):
        pltpu.sync_copy(zbuf, acc.at[pl.ds(zbase + k * SUB, SUB)])
        return carry
    lax.fori_loop(0, nfull, zcopy, 0)
    pltpu.sync_copy(zbuf.at[pl.ds(0, rem)],
                    acc.at[pl.ds(zbase + nfull * SUB, rem)])
    plsc.subcore_barrier()

    row_base = wid * (PER_W // SUB)
    e_base = wid * PER_W

    def chunk(i, carry):
        row0 = row_base + i * (CHUNK // SUB)
        pltpu.sync_copy(dst2d.at[pl.ds(row0, CHUNK // SUB)], idxd)
        pltpu.sync_copy(g_hbm.at[pl.ds(e_base + i * CHUNK, CHUNK)], rows)
        for j in range(CHUNK // SUB):
            pltpu.sync_copy(rows.at[pl.ds(j * SUB, SUB)],
                            acc.at[idxd.at[j]], add=True)
        return carry

    lax.fori_loop(0, NCHUNK, chunk, 0)
    plsc.subcore_barrier()
    pltpu.sync_copy(acc.at[pl.ds(zbase, ROWS_PER_TILE)],
                    out.at[c, pl.ds(zbase, ROWS_PER_TILE)])


def _make_scatter():
    return pl.kernel(
        _scatter_body,
        out_type=jax.ShapeDtypeStruct((NC, NP, 16), F32),
        mesh=_mesh(),
        scratch_types=[
            pltpu.VMEM_SHARED((NP, 16), F32),
            pltpu.VMEM((SUB, 16), F32),
            pltpu.VMEM((CHUNK, 16), F32),
            pltpu.VMEM((CHUNK // SUB, SUB), jnp.int32),
        ],
    )


BN = 8192
GRID_N = (NP + BN - 1) // BN
BE = 8192
GRID_E = EP // BE


def _full(shape):
    return pl.BlockSpec(shape, lambda i: (0,) * len(shape))


def _blk(w):
    return pl.BlockSpec((BN, w), lambda i: (i, 0))


def _node0_body(x_ref, wa, wb, w1, b1, t_ref, u_ref, s_ref):
    y = x_ref[...]
    a = jnp.dot(y, wa[...], preferred_element_type=F32)
    b = jnp.dot(y, wb[...], preferred_element_type=F32)
    z = jnp.zeros((BN, DTAB - 32 - y.shape[1]), F32)
    t_ref[...] = jnp.concatenate([a, y, z], axis=1)
    u_ref[...] = jnp.concatenate([b, -y, z], axis=1)
    s_ref[...] = jnp.dot(y, w1[...], preferred_element_type=F32) + b1[...]


def _node0(x_p, wa, wb, w1, b1):
    cin = x_p.shape[1]
    cout = w1.shape[1]
    return pl.pallas_call(
        _node0_body,
        grid=(GRID_N,),
        in_specs=[_blk(cin), _full((cin, 32)), _full((cin, 32)),
                  _full((cin, cout)), _full((1, cout))],
        out_specs=[_blk(DTAB), _blk(DTAB), _blk(cout)],
        out_shape=[jax.ShapeDtypeStruct((NP, DTAB), F32),
                   jax.ShapeDtypeStruct((NP, DTAB), F32),
                   jax.ShapeDtypeStruct((NP, cout), F32)],
    )(x_p, wa, wb, w1, b1)


def _node_mid_body(pa, pb, deg, skip, w2, b2, wa, wb, w1, b1,
                   t_ref, u_ref, s_ref, *, cl):
    p = pa[...] + pb[...]
    y = jax.nn.relu(jnp.dot(p[:, :cl], w2[...], preferred_element_type=F32)
                    + deg[...] * b2[...] + skip[...])
    a = jnp.dot(y, wa[...], preferred_element_type=F32)
    b = jnp.dot(y, wb[...], preferred_element_type=F32)
    t_ref[...] = jnp.concatenate([a, y], axis=1)
    u_ref[...] = jnp.concatenate([b, -y], axis=1)
    s_ref[...] = jnp.dot(y, w1[...], preferred_element_type=F32) + b1[...]


def _node_mid(pa, pb, deg, skip, w2, b2, wa, wb, w1, b1, *, cl):
    cmid = w2.shape[1]
    cnext = w1.shape[1]
    return pl.pallas_call(
        functools.partial(_node_mid_body, cl=cl),
        grid=(GRID_N,),
        in_specs=[_blk(16), _blk(16), _blk(1), _blk(cmid),
                  _full((cl, cmid)), _full((1, cmid)),
                  _full((cmid, 32)), _full((cmid, 32)),
                  _full((cmid, cnext)), _full((1, cnext))],
        out_specs=[_blk(DTAB), _blk(DTAB), _blk(cnext)],
        out_shape=[jax.ShapeDtypeStruct((NP, DTAB), F32),
                   jax.ShapeDtypeStruct((NP, DTAB), F32),
                   jax.ShapeDtypeStruct((NP, cnext), F32)],
    )(pa, pb, deg, skip, w2, b2, wa, wb, w1, b1)


def _node1_body(pa, pb, skip, w2, b2, wa, wb, w1, b1,
                t_ref, u_ref, s_ref, d_ref):
    p = pa[...] + pb[...]
    deg = p[:, 8:9]
    y = jax.nn.relu(jnp.dot(p[:, :8], w2[...], preferred_element_type=F32)
                    + deg * b2[...] + skip[...])
    a = jnp.dot(y, wa[...], preferred_element_type=F32)
    b = jnp.dot(y, wb[...], preferred_element_type=F32)
    t_ref[...] = jnp.concatenate([a, y], axis=1)
    u_ref[...] = jnp.concatenate([b, -y], axis=1)
    s_ref[...] = jnp.dot(y, w1[...], preferred_element_type=F32) + b1[...]
    d_ref[...] = deg


def _node1(pa, pb, skip, w2, b2, wa, wb, w1, b1):
    return pl.pallas_call(
        _node1_body,
        grid=(GRID_N,),
        in_specs=[_blk(16), _blk(16), _blk(16),
                  _full((8, 16)), _full((1, 16)),
                  _full((16, 32)), _full((16, 32)),
                  _full((16, 16)), _full((1, 16))],
        out_specs=[_blk(DTAB), _blk(DTAB), _blk(16), _blk(1)],
        out_shape=[jax.ShapeDtypeStruct((NP, DTAB), F32),
                   jax.ShapeDtypeStruct((NP, DTAB), F32),
                   jax.ShapeDtypeStruct((NP, 16), F32),
                   jax.ShapeDtypeStruct((NP, 1), F32)],
    )(pa, pb, skip, w2, b2, wa, wb, w1, b1)


def _node_last_body(pa, pb, deg, skip, w2, b2, o_ref):
    p = pa[...] + pb[...]
    o_ref[...] = (jnp.dot(p[:, :16], w2[...], preferred_element_type=F32)
                  + deg[...] * b2[...] + skip[...])


def _node_last(pa, pb, deg, skip, w2, b2):
    return pl.pallas_call(
        _node_last_body,
        grid=(GRID_N,),
        in_specs=[_blk(16), _blk(16), _blk(1), _blk(1),
                  _full((16, 1)), _full((1, 1))],
        out_specs=_blk(1),
        out_shape=jax.ShapeDtypeStruct((NP, 1), F32),
    )(pa, pb, deg, skip, w2, b2)


def _edge_body(g1, g2, w, wm1w, bm1, wm2, bm2, out, *, cl):
    a = g1[...]
    b = g2[...]
    wv = w[...]
    pre = (a[:, :32] + b[:, :32]
           + wv[:, 0:1] * wm1w[0:1, :] + wv[:, 1:2] * wm1w[1:2, :] + bm1[...])
    h = jnp.dot(jax.nn.relu(pre), wm2[...], preferred_element_type=F32) + bm2[...]
    g = h * (a[:, 32:32 + cl] + b[:, 32:32 + cl])
    if cl == 8:
        out[...] = jnp.concatenate(
            [g, jnp.ones((BE, 1), F32), jnp.zeros((BE, 7), F32)], axis=1)
    else:
        out[...] = g


def _edge(g1v, g2v, w_p, wm1w, bm1, wm2, bm2, *, cl):
    eb = lambda w: pl.BlockSpec((BE, w), lambda i: (i, 0))
    return pl.pallas_call(
        functools.partial(_edge_body, cl=cl),
        grid=(GRID_E,),
        in_specs=[eb(DTAB), eb(DTAB), eb(2),
                  _full((2, 32)), _full((1, 32)),
                  _full((32, cl)), _full((1, cl))],
        out_specs=eb(16),
        out_shape=jax.ShapeDtypeStruct((EP, 16), F32),
    )(g1v, g2v, w_p, wm1w, bm1, wm2, bm2)


def kernel(features, edges, weights, p_diff, p_h, p_out):
    gather = _make_gather()
    scatter = _make_scatter()

    src = edges[0]
    dst = edges[1]
    pad_e = EP - N_EDGES
    src_p = jnp.concatenate([src, jnp.full((pad_e,), N_NODES, jnp.int32)])
    dst_p = jnp.concatenate([dst, jnp.full((pad_e,), N_NODES, jnp.int32)])
    src2d = src_p.reshape(EP // SUB, SUB)
    dst2d = dst_p.reshape(EP // SUB, SUB)
    w_p = jnp.concatenate([weights, jnp.zeros((pad_e, 2), F32)])
    x_p = jnp.concatenate([features, jnp.zeros((NP - N_NODES, 8), F32)])

    def r2(v):
        return v.reshape(1, -1)

    def run_edges(t, u, p, cl):
        g1v, g2v = gather(t, u, dst2d, src2d)
        gv = _edge(g1v, g2v, w_p, p["Wm1"][2 * cl:], r2(p["bm1"]),
                   p["Wm2"], r2(p["bm2"]), cl=cl)
        ps = scatter(gv, dst2d)
        return ps[0], ps[1]

    t, u, skip = _node0(x_p, p_diff["Wm1"][:8], p_diff["Wm1"][8:16],
                        p_diff["W1"], r2(p_diff["b1"]))
    pa, pb = run_edges(t, u, p_diff, 8)
    t, u, skip, deg = _node1(pa, pb, skip,
                             p_diff["W2"], r2(p_diff["b2"]),
                             p_h["Wm1"][:16], p_h["Wm1"][16:32],
                             p_h["W1"], r2(p_h["b1"]))

    for _ in range(2):
        pa, pb = run_edges(t, u, p_h, 16)
        t, u, skip = _node_mid(pa, pb, deg, skip,
                               p_h["W2"], r2(p_h["b2"]),
                               p_h["Wm1"][:16], p_h["Wm1"][16:32],
                               p_h["W1"], r2(p_h["b1"]), cl=16)

    pa, pb = run_edges(t, u, p_h, 16)
    t, u, skip = _node_mid(pa, pb, deg, skip,
                           p_h["W2"], r2(p_h["b2"]),
                           p_out["Wm1"][:16], p_out["Wm1"][16:32],
                           p_out["W1"], r2(p_out["b1"]), cl=16)

    pa, pb = run_edges(t, u, p_out, 16)
    out = _node_last(pa, pb, deg, skip, p_out["W2"], r2(p_out["b2"]))
    return out[:N_NODES]


# speedup vs baseline: 3.6007x; 3.6007x over previous
import functools

import jax
import jax.numpy as jnp
from jax import lax
from jax.experimental import pallas as pl
from jax.experimental.pallas import tpu as pltpu
from jax.experimental.pallas import tpu_sc as plsc

N_NODES = 100000
N_EDGES = 3200000
NC = 2
NS = 16
NW = NC * NS
CHUNK = 512
SUB = 128
PER_W = 100352
EP = NW * PER_W
NCHUNK = PER_W // CHUNK
NP = 100016
ROWS_PER_TILE = NP // NS
DTAB = 48

F32 = jnp.float32


def _mesh():
    return plsc.VectorSubcoreMesh(
        core_axis_name="c", subcore_axis_name="s", num_cores=NC, num_subcores=NS
    )


def _gather_body(t_hbm, u_hbm, dst2d, src2d, g1_out, g2_out,
                 idxd, idxs, r1, r2, sem):
    wid = lax.axis_index("c") * NS + lax.axis_index("s")
    row_base = wid * (PER_W // SUB)
    e_base = wid * PER_W

    def chunk(i, carry):
        row0 = row_base + i * (CHUNK // SUB)
        pltpu.sync_copy(dst2d.at[pl.ds(row0, CHUNK // SUB)], idxd)
        pltpu.sync_copy(src2d.at[pl.ds(row0, CHUNK // SUB)], idxs)
        copies = []
        for j in range(CHUNK // SUB):
            copies.append(pltpu.async_copy(
                t_hbm.at[idxd.at[j]], r1.at[pl.ds(j * SUB, SUB)], sem))
            copies.append(pltpu.async_copy(
                u_hbm.at[idxs.at[j]], r2.at[pl.ds(j * SUB, SUB)], sem))
        for cp in copies:
            cp.wait()
        e0 = e_base + i * CHUNK
        pltpu.sync_copy(r1, g1_out.at[pl.ds(e0, CHUNK)])
        pltpu.sync_copy(r2, g2_out.at[pl.ds(e0, CHUNK)])
        return carry

    lax.fori_loop(0, NCHUNK, chunk, 0)


def _make_gather():
    return pl.kernel(
        _gather_body,
        out_type=[
            jax.ShapeDtypeStruct((EP, DTAB), F32),
            jax.ShapeDtypeStruct((EP, DTAB), F32),
        ],
        mesh=_mesh(),
        scratch_types=[
            pltpu.VMEM((CHUNK // SUB, SUB), jnp.int32),
            pltpu.VMEM((CHUNK // SUB, SUB), jnp.int32),
            pltpu.VMEM((CHUNK, DTAB), F32),
            pltpu.VMEM((CHUNK, DTAB), F32),
            pltpu.SemaphoreType.DMA,
        ],
        compiler_params=pltpu.CompilerParams(use_tc_tiling_on_sc=False),
    )


def _scatter_body(g_hbm, dst2d, out, acc, zbuf, rows, idxd):
    c = lax.axis_index("c")
    s = lax.axis_index("s")
    wid = c * NS + s

    def zrow(i, carry):
        zbuf[i, :] = jnp.zeros((16,), F32)
        return carry
    lax.fori_loop(0, SUB, zrow, 0)
    zbase = s * ROWS_PER_TILE
    nfull = ROWS_PER_TILE // SUB
    rem = ROWS_PER_TILE - nfull * SUB

    def zcopy(k, carry):
        pltpu.sync_copy(zbuf, acc.at[pl.ds(zbase + k * SUB, SUB)])
        return carry
    lax.fori_loop(0, nfull, zcopy, 0)
    pltpu.sync_copy(zbuf.at[pl.ds(0, rem)],
                    acc.at[pl.ds(zbase + nfull * SUB, rem)])
    plsc.subcore_barrier()

    row_base = wid * (PER_W // SUB)
    e_base = wid * PER_W

    def chunk(i, carry):
        row0 = row_base + i * (CHUNK // SUB)
        pltpu.sync_copy(dst2d.at[pl.ds(row0, CHUNK // SUB)], idxd)
        pltpu.sync_copy(g_hbm.at[pl.ds(e_base + i * CHUNK, CHUNK)], rows)
        for j in range(CHUNK // SUB):
            pltpu.sync_copy(rows.at[pl.ds(j * SUB, SUB)],
                            acc.at[idxd.at[j]], add=True)
        return carry

    lax.fori_loop(0, NCHUNK, chunk, 0)
    plsc.subcore_barrier()
    pltpu.sync_copy(acc.at[pl.ds(zbase, ROWS_PER_TILE)],
                    out.at[c, pl.ds(zbase, ROWS_PER_TILE)])


def _make_scatter():
    return pl.kernel(
        _scatter_body,
        out_type=jax.ShapeDtypeStruct((NC, NP, 16), F32),
        mesh=_mesh(),
        scratch_types=[
            pltpu.VMEM_SHARED((NP, 16), F32),
            pltpu.VMEM((SUB, 16), F32),
            pltpu.VMEM((CHUNK, 16), F32),
            pltpu.VMEM((CHUNK // SUB, SUB), jnp.int32),
        ],
        compiler_params=pltpu.CompilerParams(use_tc_tiling_on_sc=False),
    )


BN = 4096
GRID_N = (NP + BN - 1) // BN
BE = 4096
GRID_E = EP // BE


def _full(shape):
    return pl.BlockSpec(shape, lambda i: (0,) * len(shape))


def _blk(w):
    return pl.BlockSpec((BN, w), lambda i: (i, 0))


def _node0_body(x_ref, wa, wb, w1, b1, t_ref, u_ref, s_ref):
    y = x_ref[...]
    a = jnp.dot(y, wa[...], preferred_element_type=F32)
    b = jnp.dot(y, wb[...], preferred_element_type=F32)
    z = jnp.zeros((BN, DTAB - 32 - y.shape[1]), F32)
    t_ref[...] = jnp.concatenate([a, y, z], axis=1)
    u_ref[...] = jnp.concatenate([b, -y, z], axis=1)
    s_ref[...] = jnp.dot(y, w1[...], preferred_element_type=F32) + b1[...]


def _node0(x_p, wa, wb, w1, b1):
    cin = x_p.shape[1]
    cout = w1.shape[1]
    return pl.pallas_call(
        _node0_body,
        grid=(GRID_N,),
        in_specs=[_blk(cin), _full((cin, 32)), _full((cin, 32)),
                  _full((cin, cout)), _full((1, cout))],
        out_specs=[_blk(DTAB), _blk(DTAB), _blk(cout)],
        out_shape=[jax.ShapeDtypeStruct((NP, DTAB), F32),
                   jax.ShapeDtypeStruct((NP, DTAB), F32),
                   jax.ShapeDtypeStruct((NP, cout), F32)],
    )(x_p, wa, wb, w1, b1)


def _node_mid_body(pa, pb, deg, skip, w2, b2, wa, wb, w1, b1,
                   t_ref, u_ref, s_ref, *, cl):
    p = pa[...] + pb[...]
    y = jax.nn.relu(jnp.dot(p[:, :cl], w2[...], preferred_element_type=F32)
                    + deg[...] * b2[...] + skip[...])
    a = jnp.dot(y, wa[...], preferred_element_type=F32)
    b = jnp.dot(y, wb[...], preferred_element_type=F32)
    t_ref[...] = jnp.concatenate([a, y], axis=1)
    u_ref[...] = jnp.concatenate([b, -y], axis=1)
    s_ref[...] = jnp.dot(y, w1[...], preferred_element_type=F32) + b1[...]


def _node_mid(pa, pb, deg, skip, w2, b2, wa, wb, w1, b1, *, cl):
    cmid = w2.shape[1]
    cnext = w1.shape[1]
    return pl.pallas_call(
        functools.partial(_node_mid_body, cl=cl),
        grid=(GRID_N,),
        in_specs=[_blk(16), _blk(16), _blk(1), _blk(cmid),
                  _full((cl, cmid)), _full((1, cmid)),
                  _full((cmid, 32)), _full((cmid, 32)),
                  _full((cmid, cnext)), _full((1, cnext))],
        out_specs=[_blk(DTAB), _blk(DTAB), _blk(cnext)],
        out_shape=[jax.ShapeDtypeStruct((NP, DTAB), F32),
                   jax.ShapeDtypeStruct((NP, DTAB), F32),
                   jax.ShapeDtypeStruct((NP, cnext), F32)],
    )(pa, pb, deg, skip, w2, b2, wa, wb, w1, b1)


def _node1_body(pa, pb, skip, w2, b2, wa, wb, w1, b1,
                t_ref, u_ref, s_ref, d_ref):
    p = pa[...] + pb[...]
    deg = p[:, 8:9]
    y = jax.nn.relu(jnp.dot(p[:, :8], w2[...], preferred_element_type=F32)
                    + deg * b2[...] + skip[...])
    a = jnp.dot(y, wa[...], preferred_element_type=F32)
    b = jnp.dot(y, wb[...], preferred_element_type=F32)
    t_ref[...] = jnp.concatenate([a, y], axis=1)
    u_ref[...] = jnp.concatenate([b, -y], axis=1)
    s_ref[...] = jnp.dot(y, w1[...], preferred_element_type=F32) + b1[...]
    d_ref[...] = deg


def _node1(pa, pb, skip, w2, b2, wa, wb, w1, b1):
    return pl.pallas_call(
        _node1_body,
        grid=(GRID_N,),
        in_specs=[_blk(16), _blk(16), _blk(16),
                  _full((8, 16)), _full((1, 16)),
                  _full((16, 32)), _full((16, 32)),
                  _full((16, 16)), _full((1, 16))],
        out_specs=[_blk(DTAB), _blk(DTAB), _blk(16), _blk(1)],
        out_shape=[jax.ShapeDtypeStruct((NP, DTAB), F32),
                   jax.ShapeDtypeStruct((NP, DTAB), F32),
                   jax.ShapeDtypeStruct((NP, 16), F32),
                   jax.ShapeDtypeStruct((NP, 1), F32)],
    )(pa, pb, skip, w2, b2, wa, wb, w1, b1)


def _node_last_body(pa, pb, deg, skip, w2, b2, o_ref):
    p = pa[...] + pb[...]
    o_ref[...] = (jnp.dot(p[:, :16], w2[...], preferred_element_type=F32)
                  + deg[...] * b2[...] + skip[...])


def _node_last(pa, pb, deg, skip, w2, b2):
    return pl.pallas_call(
        _node_last_body,
        grid=(GRID_N,),
        in_specs=[_blk(16), _blk(16), _blk(1), _blk(1),
                  _full((16, 1)), _full((1, 1))],
        out_specs=_blk(1),
        out_shape=jax.ShapeDtypeStruct((NP, 1), F32),
    )(pa, pb, deg, skip, w2, b2)


def _edge_body(g1, g2, w, wm1w, bm1, wm2, bm2, out, *, cl):
    a = g1[...]
    b = g2[...]
    wv = w[...]
    pre = (a[:, :32] + b[:, :32]
           + wv[:, 0:1] * wm1w[0:1, :] + wv[:, 1:2] * wm1w[1:2, :] + bm1[...])
    h = jnp.dot(jax.nn.relu(pre), wm2[...], preferred_element_type=F32) + bm2[...]
    g = h * (a[:, 32:32 + cl] + b[:, 32:32 + cl])
    if cl == 8:
        out[...] = jnp.concatenate(
            [g, jnp.ones((BE, 1), F32), jnp.zeros((BE, 7), F32)], axis=1)
    else:
        out[...] = g


def _edge(g1v, g2v, w_p, wm1w, bm1, wm2, bm2, *, cl):
    eb = lambda w: pl.BlockSpec((BE, w), lambda i: (i, 0))
    return pl.pallas_call(
        functools.partial(_edge_body, cl=cl),
        grid=(GRID_E,),
        in_specs=[eb(DTAB), eb(DTAB), eb(2),
                  _full((2, 32)), _full((1, 32)),
                  _full((32, cl)), _full((1, cl))],
        out_specs=eb(16),
        out_shape=jax.ShapeDtypeStruct((EP, 16), F32),
    )(g1v, g2v, w_p, wm1w, bm1, wm2, bm2)


def kernel(features, edges, weights, p_diff, p_h, p_out):
    gather = _make_gather()
    scatter = _make_scatter()

    src = edges[0]
    dst = edges[1]
    pad_e = EP - N_EDGES
    src_p = jnp.concatenate([src, jnp.full((pad_e,), N_NODES, jnp.int32)])
    dst_p = jnp.concatenate([dst, jnp.full((pad_e,), N_NODES, jnp.int32)])
    src2d = src_p.reshape(EP // SUB, SUB)
    dst2d = dst_p.reshape(EP // SUB, SUB)
    w_p = jnp.concatenate([weights, jnp.zeros((pad_e, 2), F32)])
    x_p = jnp.concatenate([features, jnp.zeros((NP - N_NODES, 8), F32)])

    def r2(v):
        return v.reshape(1, -1)

    def run_edges(t, u, p, cl):
        g1v, g2v = gather(t, u, dst2d, src2d)
        gv = _edge(g1v, g2v, w_p, p["Wm1"][2 * cl:], r2(p["bm1"]),
                   p["Wm2"], r2(p["bm2"]), cl=cl)
        ps = scatter(gv, dst2d)
        return ps[0], ps[1]

    t, u, skip = _node0(x_p, p_diff["Wm1"][:8], p_diff["Wm1"][8:16],
                        p_diff["W1"], r2(p_diff["b1"]))
    pa, pb = run_edges(t, u, p_diff, 8)
    t, u, skip, deg = _node1(pa, pb, skip,
                             p_diff["W2"], r2(p_diff["b2"]),
                             p_h["Wm1"][:16], p_h["Wm1"][16:32],
                             p_h["W1"], r2(p_h["b1"]))

    for _ in range(2):
        pa, pb = run_edges(t, u, p_h, 16)
        t, u, skip = _node_mid(pa, pb, deg, skip,
                               p_h["W2"], r2(p_h["b2"]),
                               p_h["Wm1"][:16], p_h["Wm1"][16:32],
                               p_h["W1"], r2(p_h["b1"]), cl=16)

    pa, pb = run_edges(t, u, p_h, 16)
    t, u, skip = _node_mid(pa, pb, deg, skip,
                           p_h["W2"], r2(p_h["b2"]),
                           p_out["Wm1"][:16], p_out["Wm1"][16:32],
                           p_out["W1"], r2(p_out["b1"]), cl=16)

    pa, pb = run_edges(t, u, p_out, 16)
    out = _node_last(pa, pb, deg, skip, p_out["W2"], r2(p_out["b2"]))
    return out[:N_NODES]
